# baseline (speedup 1.0000x reference)
import functools

import jax
import jax.numpy as jnp
from jax import lax
from jax.experimental import pallas as pl
from jax.experimental.pallas import tpu as pltpu
from jax.experimental.pallas import tpu_sc as plsc

RANGE = 15
GRANU = 16
L = RANGE * GRANU
LANES = 16

NUM_CORES = 2
NUM_SUBCORES = 16
NW = NUM_CORES * NUM_SUBCORES

CHUNK = 25088
NBUF = 2


def _sc_quant_lookup(n_elems):
    per_w = n_elems // NW
    n_chunks = per_w // CHUNK
    n_grp = n_chunks // NBUF
    assert per_w % CHUNK == 0 and n_chunks % NBUF == 0
    assert n_elems == per_w * NW

    mesh = plsc.VectorSubcoreMesh(core_axis_name="c", subcore_axis_name="s")

    @functools.partial(
        pl.kernel,
        mesh=mesh,
        compiler_params=pltpu.CompilerParams(needs_layout_passes=False),
        out_type=jax.ShapeDtypeStruct((n_elems,), jnp.float32),
        scratch_types=[
            pltpu.VMEM((RANGE, GRANU), jnp.float32),
            pltpu.VMEM((LANES,), jnp.float32),
            pltpu.VMEM((256,), jnp.float32),
            [pltpu.VMEM((CHUNK,), jnp.float32) for _ in range(NBUF)],
            [pltpu.VMEM((CHUNK,), jnp.float32) for _ in range(NBUF)],
            [pltpu.SemaphoreType.DMA for _ in range(NBUF)],
            [pltpu.SemaphoreType.DMA for _ in range(NBUF)],
        ],
    )
    def body(x_hbm, table_hbm, slog_hbm, out_hbm, rows_v, slog_v, d_v,
             inb, outb, sin, sout):
        cid = lax.axis_index("c")
        sid = lax.axis_index("s")
        wid = sid * NUM_CORES + cid
        base = wid * per_w

        pltpu.sync_copy(table_hbm, rows_v)
        pltpu.sync_copy(slog_hbm, slog_v)

        sl = slog_v[...]
        scale = jnp.exp(sl)
        kmul = jnp.float32(L) / scale
        fac = scale / jnp.float32(RANGE)

        carry = jnp.zeros((LANES,), jnp.float32)
        for r in range(RANGE):
            row = rows_v[r]
            m = jnp.max(row)
            e = jnp.exp(row - m)
            p = e / jnp.sum(e)
            cum = plsc.cumsum(p) + carry
            d_v[pl.ds(r * GRANU, GRANU)] = (cum - p) * fac
            carry = jnp.full((LANES,), jnp.max(cum), jnp.float32)
        d_v[pl.ds(L, GRANU)] = carry * fac

        def in_copy(c, b):
            off = base + c * CHUNK
            return pltpu.make_async_copy(
                x_hbm.at[pl.ds(off, CHUNK)], inb[b], sin[b])

        def out_copy(c, b):
            off = base + c * CHUNK
            return pltpu.make_async_copy(
                outb[b], out_hbm.at[pl.ds(off, CHUNK)], sout[b])

        for b in range(NBUF):
            in_copy(b, b).start()

        def grp(g, _):
            for b in range(NBUF):
                c = g * NBUF + b
                in_copy(c, b).wait()

                @pl.when(g > 0)
                def _wait_out():
                    out_copy(c - NBUF, b).wait()

                @plsc.parallel_loop(0, CHUNK, LANES, unroll=32)
                def vec_body(i):
                    v = inb[b][pl.ds(i, LANES)]
                    t = v * kmul + jnp.float32(0.5)
                    t = jnp.minimum(jnp.maximum(t, jnp.float32(0.0)),
                                    jnp.float32(L + 0.9))
                    j = t.astype(jnp.int32)
                    outb[b][pl.ds(i, LANES)] = plsc.load_gather(d_v, [j])

                out_copy(c, b).start()

                @pl.when(g < n_grp - 1)
                def _next_in():
                    in_copy(c + NBUF, b).start()
            return _

        lax.fori_loop(0, n_grp, grp, None)
        for b in range(NBUF):
            out_copy(n_chunks - NBUF + b, b).wait()

    return body


def kernel(x, table, scale_log):
    n = x.size
    s0, s1, s2, s3 = x.shape
    c_hi, c_lo = s3 // 8, 8
    ch_hi, ch_lo = s1 // 128, 128
    xt = (jnp.transpose(x, (0, 2, 3, 1))
          .reshape(s0, s2, c_hi, c_lo, ch_hi, ch_lo)
          .transpose(0, 1, 2, 4, 3, 5))
    x_flat = xt.reshape(n)
    slog = jnp.broadcast_to(scale_log.astype(jnp.float32), (LANES,))
    out = _sc_quant_lookup(n)(x_flat, table, slog)
    out = (out.reshape(s0, s2, c_hi, ch_hi, c_lo, ch_lo)
           .transpose(0, 1, 2, 4, 3, 5)
           .reshape(s0, s2, s3, s1))
    return jnp.transpose(out, (0, 3, 1, 2))


# speedup vs baseline: 1.0786x; 1.0786x over previous
import functools

import jax
import jax.numpy as jnp
from jax import lax
from jax.experimental import pallas as pl
from jax.experimental.pallas import tpu as pltpu
from jax.experimental.pallas import tpu_sc as plsc

RANGE = 15
GRANU = 16
L = RANGE * GRANU
LANES = 16

NUM_CORES = 2
NUM_SUBCORES = 16
NW = NUM_CORES * NUM_SUBCORES

CHUNK = 25088
NBUF = 2


def _sc_quant_lookup(n_elems):
    per_w = n_elems // NW
    n_chunks = per_w // CHUNK
    n_grp = n_chunks // NBUF
    assert per_w % CHUNK == 0 and n_chunks % NBUF == 0
    assert n_elems == per_w * NW

    mesh = plsc.VectorSubcoreMesh(core_axis_name="c", subcore_axis_name="s")

    @functools.partial(
        pl.kernel,
        mesh=mesh,
        compiler_params=pltpu.CompilerParams(needs_layout_passes=False),
        out_type=jax.ShapeDtypeStruct((n_elems,), jnp.float32),
        scratch_types=[
            pltpu.VMEM((RANGE, GRANU), jnp.float32),
            pltpu.VMEM((LANES,), jnp.float32),
            pltpu.VMEM((256,), jnp.float32),
            [pltpu.VMEM((CHUNK,), jnp.float32) for _ in range(NBUF)],
            [pltpu.VMEM((CHUNK,), jnp.float32) for _ in range(NBUF)],
            [pltpu.SemaphoreType.DMA for _ in range(NBUF)],
            [pltpu.SemaphoreType.DMA for _ in range(NBUF)],
        ],
    )
    def body(x_hbm, table_hbm, slog_hbm, out_hbm, rows_v, slog_v, d_v,
             inb, outb, sin, sout):
        cid = lax.axis_index("c")
        sid = lax.axis_index("s")
        wid = sid * NUM_CORES + cid
        base = wid * per_w

        pltpu.sync_copy(table_hbm, rows_v)
        pltpu.sync_copy(slog_hbm, slog_v)

        sl = slog_v[...]
        scale = jnp.exp(sl)
        kmul = jnp.float32(L) / scale
        fac = scale / jnp.float32(RANGE)

        carry = jnp.zeros((LANES,), jnp.float32)
        for r in range(RANGE):
            row = rows_v[r]
            m = jnp.max(row)
            e = jnp.exp(row - m)
            p = e / jnp.sum(e)
            cum = plsc.cumsum(p) + carry
            d_v[pl.ds(r * GRANU, GRANU)] = (cum - p) * fac
            carry = jnp.full((LANES,), jnp.max(cum), jnp.float32)
        d_v[pl.ds(L, GRANU)] = carry * fac

        def in_copy(c, b):
            off = base + c * CHUNK
            return pltpu.make_async_copy(
                x_hbm.at[pl.ds(off, CHUNK)], inb[b], sin[b])

        def out_copy(c, b):
            off = base + c * CHUNK
            return pltpu.make_async_copy(
                outb[b], out_hbm.at[pl.ds(off, CHUNK)], sout[b])

        for b in range(NBUF):
            in_copy(b, b).start()

        def grp(g, _):
            for b in range(NBUF):
                c = g * NBUF + b
                in_copy(c, b).wait()

                @pl.when(g > 0)
                def _wait_out():
                    out_copy(c - NBUF, b).wait()

                @plsc.parallel_loop(0, CHUNK, LANES, unroll=16)
                def vec_body(i):
                    v = inb[b][pl.ds(i, LANES)]
                    t = v * kmul + jnp.float32(0.5)
                    t = jnp.minimum(jnp.maximum(t, jnp.float32(0.0)),
                                    jnp.float32(L + 0.9))
                    j = t.astype(jnp.int32)
                    outb[b][pl.ds(i, LANES)] = plsc.load_gather(d_v, [j])

                out_copy(c, b).start()

                @pl.when(g < n_grp - 1)
                def _next_in():
                    in_copy(c + NBUF, b).start()
            return _

        lax.fori_loop(0, n_grp, grp, None)
        for b in range(NBUF):
            out_copy(n_chunks - NBUF + b, b).wait()

    return body


def kernel(x, table, scale_log):
    n = x.size
    s0, s1, s2, s3 = x.shape
    c_hi, c_lo = s3 // 8, 8
    ch_hi, ch_lo = s1 // 128, 128
    xt = (jnp.transpose(x, (0, 2, 3, 1))
          .reshape(s0, s2, c_hi, c_lo, ch_hi, ch_lo)
          .transpose(0, 1, 2, 4, 3, 5))
    x_flat = xt.reshape(n)
    slog = jnp.broadcast_to(scale_log.astype(jnp.float32), (LANES,))
    out = _sc_quant_lookup(n)(x_flat, table, slog)
    out = (out.reshape(s0, s2, c_hi, ch_hi, c_lo, ch_lo)
           .transpose(0, 1, 2, 4, 3, 5)
           .reshape(s0, s2, s3, s1))
    return jnp.transpose(out, (0, 3, 1, 2))
